# baseline (speedup 1.0000x reference)
import functools

import jax
import jax.numpy as jnp
from jax import lax
from jax.experimental import pallas as pl
from jax.experimental.pallas import tpu as pltpu
from jax.experimental.pallas import tpu_sc as plsc

_LANES = 128
_RING = 16
_SUB = 8


@functools.lru_cache(maxsize=None)
def _build_gather(B, V, D):
    info = plsc.get_sparse_core_info()
    NC, NS = info.num_cores, info.num_subcores
    NW = NC * NS
    bpw = B // NW

    mesh = plsc.VectorSubcoreMesh(core_axis_name="c", subcore_axis_name="s")

    @functools.partial(
        pl.kernel,
        mesh=mesh,
        out_type=jax.ShapeDtypeStruct((D, B), jnp.float32),
        compiler_params=pltpu.CompilerParams(needs_layout_passes=False),
        scratch_types=[
            pltpu.VMEM((bpw,), jnp.int32),
            pltpu.VMEM((_RING, D, _LANES), jnp.float32),
            pltpu.VMEM((D, bpw), jnp.float32),
            pltpu.SemaphoreType.DMA,
        ],
    )
    def gather_kernel(table_hbm, idx_hbm, out_hbm, idx_v, win_v, vals_v, sem):
        wid = lax.axis_index("s") * NC + lax.axis_index("c")
        base = wid * bpw
        pltpu.sync_copy(idx_hbm.at[pl.ds(base, bpw)], idx_v)

        iota16 = lax.iota(jnp.int32, 16)
        zeros16 = iota16 * 0
        rows = [iota16 + 16 * g for g in range(D // 16)]

        def wait_one(_):
            pltpu.make_async_copy(
                table_hbm.at[:, pl.ds(0, _LANES)],
                win_v.at[0],
                sem,
            ).wait()

        @pl.loop(0, bpw // 16, init_carry=(jnp.int32(0), jnp.int32(-1)))
        def _(g, carry):
            slot, w_prev = carry
            j0 = g * 16
            vec = idx_v[pl.ds(j0, 16)]
            cols = [vec[k] for k in range(16)]
            for half in range(16 // _SUB):
                ks = range(half * _SUB, (half + 1) * _SUB)
                slots = []
                nfetch = jnp.int32(0)
                for k in ks:
                    w = lax.shift_right_logical(cols[k], 7)
                    is_new = (w != w_prev).astype(jnp.int32)
                    slot = slot + is_new
                    nfetch = nfetch + is_new
                    ring_slot = lax.rem(slot, _RING)

                    @pl.when(w != w_prev)
                    def _():
                        pltpu.async_copy(
                            table_hbm.at[:, pl.ds(w * _LANES, _LANES)],
                            win_v.at[ring_slot],
                            sem,
                        )

                    slots.append(ring_slot)
                    w_prev = w
                pl.loop(0, nfetch)(wait_one)
                for k, sl in zip(ks, slots):
                    lane = zeros16 + (cols[k] & (_LANES - 1))
                    slotv = zeros16 + sl
                    col_j = zeros16 + (j0 + k)
                    for r in rows:
                        vals = plsc.load_gather(win_v, [slotv, r, lane])
                        plsc.store_scatter(vals_v, [r, col_j], vals)
            return slot, w_prev

        pltpu.sync_copy(vals_v, out_hbm.at[:, pl.ds(base, bpw)])

    return gather_kernel


@functools.lru_cache(maxsize=None)
def _build_unpermute(B, D):
    info = plsc.get_sparse_core_info()
    NC, NS = info.num_cores, info.num_subcores
    NW = NC * NS
    bpw = B // NW
    cpw = bpw // _LANES

    mesh = plsc.VectorSubcoreMesh(core_axis_name="c", subcore_axis_name="s")

    @functools.partial(
        pl.kernel,
        mesh=mesh,
        out_type=jax.ShapeDtypeStruct((B, D), jnp.float32),
        compiler_params=pltpu.CompilerParams(
            use_tc_tiling_on_sc=False, needs_layout_passes=False),
        scratch_types=[
            pltpu.VMEM((cpw, _LANES), jnp.int32),
            pltpu.VMEM((D, _LANES), jnp.float32),
            pltpu.VMEM((2, _LANES, D), jnp.float32),
            pltpu.SemaphoreType.DMA,
        ],
    )
    def unpermute_kernel(vals_hbm, pos_hbm, out_hbm, pos_v, win_v, rows_v, sem):
        wid = lax.axis_index("s") * NC + lax.axis_index("c")
        pltpu.sync_copy(pos_hbm.at[pl.ds(wid * cpw, cpw)], pos_v)

        iota16 = lax.iota(jnp.int32, 16)
        zeros16 = iota16 * 0
        rows = [iota16 + 16 * g for g in range(D // 16)]

        for c in range(cpw):
            buf = c % 2
            pltpu.sync_copy(
                vals_hbm.at[:, pl.ds(wid * bpw + c * _LANES, _LANES)],
                win_v,
            )
            if c >= 2:
                pltpu.make_async_copy(
                    rows_v.at[buf], out_hbm.at[pos_v.at[c]], sem
                ).wait()
            for l in range(_LANES):
                lane = zeros16 + l
                for gidx, r in enumerate(rows):
                    vals = plsc.load_gather(win_v, [r, lane])
                    rows_v[buf, l, pl.ds(gidx * 16, 16)] = vals
            pltpu.async_copy(rows_v.at[buf], out_hbm.at[pos_v.at[c]], sem)
        for c in range(max(cpw - 2, 0), cpw):
            pltpu.make_async_copy(
                rows_v.at[c % 2], out_hbm.at[pos_v.at[c]], sem
            ).wait()

    return unpermute_kernel


def kernel(x, style_shift_weight):
    B, = x.shape
    V, D = style_shift_weight.shape
    idx = x.astype(jnp.int32)
    sorted_idx, pos = lax.sort((idx, lax.iota(jnp.int32, B)), num_keys=1)
    vals_sorted = _build_gather(B, V, D)(style_shift_weight.T, sorted_idx)
    pos2d = pos.reshape(B // _LANES, _LANES)
    return _build_unpermute(B, D)(vals_sorted, pos2d)


# speedup vs baseline: 1.1472x; 1.1472x over previous
import functools

import jax
import jax.numpy as jnp
from jax import lax
from jax.experimental import pallas as pl
from jax.experimental.pallas import tpu as pltpu
from jax.experimental.pallas import tpu_sc as plsc

_LANES = 128
_RING = 24
_SUB = 16


@functools.lru_cache(maxsize=None)
def _build_gather(B, V, D):
    info = plsc.get_sparse_core_info()
    NC, NS = info.num_cores, info.num_subcores
    NW = NC * NS
    bpw = B // NW

    mesh = plsc.VectorSubcoreMesh(core_axis_name="c", subcore_axis_name="s")

    @functools.partial(
        pl.kernel,
        mesh=mesh,
        out_type=jax.ShapeDtypeStruct((D, B), jnp.float32),
        compiler_params=pltpu.CompilerParams(needs_layout_passes=False),
        scratch_types=[
            pltpu.VMEM((bpw,), jnp.int32),
            pltpu.VMEM((_RING, D, _LANES), jnp.float32),
            pltpu.VMEM((D, bpw), jnp.float32),
            pltpu.SemaphoreType.DMA,
        ],
    )
    def gather_kernel(table_hbm, idx_hbm, out_hbm, idx_v, win_v, vals_v, sem):
        wid = lax.axis_index("s") * NC + lax.axis_index("c")
        base = wid * bpw
        pltpu.sync_copy(idx_hbm.at[pl.ds(base, bpw)], idx_v)

        iota16 = lax.iota(jnp.int32, 16)
        zeros16 = iota16 * 0
        rows = [iota16 + 16 * g for g in range(D // 16)]

        def wait_one(_):
            pltpu.make_async_copy(
                table_hbm.at[:, pl.ds(0, _LANES)],
                win_v.at[0],
                sem,
            ).wait()

        @pl.loop(0, bpw // 16, init_carry=(jnp.int32(0), jnp.int32(-1)))
        def _(g, carry):
            slot, w_prev = carry
            j0 = g * 16
            vec = idx_v[pl.ds(j0, 16)]
            cols = [vec[k] for k in range(16)]
            for half in range(16 // _SUB):
                ks = range(half * _SUB, (half + 1) * _SUB)
                slots = []
                nfetch = jnp.int32(0)
                for k in ks:
                    w = lax.shift_right_logical(cols[k], 7)
                    is_new = (w != w_prev).astype(jnp.int32)
                    slot = slot + is_new
                    nfetch = nfetch + is_new
                    ring_slot = lax.rem(slot, _RING)

                    @pl.when(w != w_prev)
                    def _():
                        pltpu.async_copy(
                            table_hbm.at[:, pl.ds(w * _LANES, _LANES)],
                            win_v.at[ring_slot],
                            sem,
                        )

                    slots.append(ring_slot)
                    w_prev = w
                pl.loop(0, nfetch)(wait_one)
                for k, sl in zip(ks, slots):
                    lane = zeros16 + (cols[k] & (_LANES - 1))
                    slotv = zeros16 + sl
                    col_j = zeros16 + (j0 + k)
                    for r in rows:
                        vals = plsc.load_gather(win_v, [slotv, r, lane])
                        plsc.store_scatter(vals_v, [r, col_j], vals)
            return slot, w_prev

        pltpu.sync_copy(vals_v, out_hbm.at[:, pl.ds(base, bpw)])

    return gather_kernel


@functools.lru_cache(maxsize=None)
def _build_unpermute(B, D):
    info = plsc.get_sparse_core_info()
    NC, NS = info.num_cores, info.num_subcores
    NW = NC * NS
    bpw = B // NW
    cpw = bpw // _LANES

    mesh = plsc.VectorSubcoreMesh(core_axis_name="c", subcore_axis_name="s")

    @functools.partial(
        pl.kernel,
        mesh=mesh,
        out_type=jax.ShapeDtypeStruct((B, D), jnp.float32),
        compiler_params=pltpu.CompilerParams(
            use_tc_tiling_on_sc=False, needs_layout_passes=False),
        scratch_types=[
            pltpu.VMEM((cpw, _LANES), jnp.int32),
            pltpu.VMEM((D, _LANES), jnp.float32),
            pltpu.VMEM((2, _LANES, D), jnp.float32),
            pltpu.SemaphoreType.DMA,
        ],
    )
    def unpermute_kernel(vals_hbm, pos_hbm, out_hbm, pos_v, win_v, rows_v, sem):
        wid = lax.axis_index("s") * NC + lax.axis_index("c")
        pltpu.sync_copy(pos_hbm.at[pl.ds(wid * cpw, cpw)], pos_v)

        iota16 = lax.iota(jnp.int32, 16)
        zeros16 = iota16 * 0
        rows = [iota16 + 16 * g for g in range(D // 16)]

        for c in range(cpw):
            buf = c % 2
            pltpu.sync_copy(
                vals_hbm.at[:, pl.ds(wid * bpw + c * _LANES, _LANES)],
                win_v,
            )
            if c >= 2:
                pltpu.make_async_copy(
                    rows_v.at[buf], out_hbm.at[pos_v.at[c]], sem
                ).wait()
            for l in range(_LANES):
                lane = zeros16 + l
                for gidx, r in enumerate(rows):
                    vals = plsc.load_gather(win_v, [r, lane])
                    rows_v[buf, l, pl.ds(gidx * 16, 16)] = vals
            pltpu.async_copy(rows_v.at[buf], out_hbm.at[pos_v.at[c]], sem)
        for c in range(max(cpw - 2, 0), cpw):
            pltpu.make_async_copy(
                rows_v.at[c % 2], out_hbm.at[pos_v.at[c]], sem
            ).wait()

    return unpermute_kernel


def kernel(x, style_shift_weight):
    B, = x.shape
    V, D = style_shift_weight.shape
    idx = x.astype(jnp.int32)
    sorted_idx, pos = lax.sort((idx, lax.iota(jnp.int32, B)), num_keys=1)
    vals_sorted = _build_gather(B, V, D)(style_shift_weight.T, sorted_idx)
    pos2d = pos.reshape(B // _LANES, _LANES)
    return _build_unpermute(B, D)(vals_sorted, pos2d)
